# baseline (speedup 1.0000x reference)
import jax
import jax.numpy as jnp
from jax.experimental import pallas as pl

_BLK = 1024


def _stage1(te_ref, w1_ref, b1_ref, w2_ref, b2_ref, lab_ref, ones8_ref,
            onesc_ref, t_ref, agg_ref, labtr_ref):
    h = jnp.dot(te_ref[...], w1_ref[...], preferred_element_type=jnp.float32)
    h = jax.nn.silu(h + b1_ref[...])
    u = jnp.dot(h, w2_ref[...], preferred_element_type=jnp.float32)
    t_ref[...] = 1.0 - jax.nn.sigmoid(u + b2_ref[...])

    lab = lab_ref[...]
    c = lab.shape[1]
    labf1 = (lab == 1).astype(jnp.float32)
    count1 = jnp.dot(labf1, onesc_ref[...], preferred_element_type=jnp.float32)
    init = (count1 > (c - count1)).astype(jnp.int32)
    eqf = (lab == init).astype(jnp.float32)
    agg_ref[0] = jnp.dot(ones8_ref[...], eqf, preferred_element_type=jnp.float32)
    lab3 = lab.reshape(8, _BLK // 8, c)
    labtr_ref[0] = jnp.transpose(lab3, (2, 0, 1))


def _stage2(labtr_ref, relt_ref, t3_ref, relrow_ref, cur_ref, relb_ref):
    t = t3_ref[0]
    c = labtr_ref.shape[1]
    labt = labtr_ref[0]
    nblk = 4
    blk = c // nblk

    s0 = None
    s1 = None
    for j in range(nblk):
        a1 = None
        a0 = None
        for k in range(blk):
            col = j * blk + k
            w = relt_ref[col] * t
            term1 = jnp.where(labt[col] == 1, w, 0.0)
            term0 = w - term1
            a1 = term1 if a1 is None else a1 + term1
            a0 = term0 if a0 is None else a0 + term0
        s1 = a1 if s1 is None else s1 + a1
        s0 = a0 if s0 is None else s0 + a0
    cur_ref[0] = (s1 > s0).astype(jnp.int32)

    relrow = relrow_ref[0:1, :]
    relb_ref[...] = jnp.broadcast_to(relrow, relb_ref.shape)


def kernel(task_embeddings, contributor_ids, contributor_labels, W1, b1, W2, b2):
    del contributor_ids
    b, hidden = task_embeddings.shape
    c = contributor_labels.shape[1]
    hh = W1.shape[1]
    nb = b // _BLK

    ones8 = jnp.ones((8, _BLK), jnp.float32)
    onesc = jnp.ones((c, 1), jnp.float32)

    t_out, agg, labtr = pl.pallas_call(
        _stage1,
        grid=(nb,),
        in_specs=[
            pl.BlockSpec((_BLK, hidden), lambda i: (i, 0)),
            pl.BlockSpec((hidden, hh), lambda i: (0, 0)),
            pl.BlockSpec((1, hh), lambda i: (0, 0)),
            pl.BlockSpec((hh, 1), lambda i: (0, 0)),
            pl.BlockSpec((1, 1), lambda i: (0, 0)),
            pl.BlockSpec((_BLK, c), lambda i: (i, 0)),
            pl.BlockSpec((8, _BLK), lambda i: (0, 0)),
            pl.BlockSpec((c, 1), lambda i: (0, 0)),
        ],
        out_specs=[
            pl.BlockSpec((_BLK, 1), lambda i: (i, 0)),
            pl.BlockSpec((1, 8, c), lambda i: (i, 0, 0)),
            pl.BlockSpec((1, c, 8, _BLK // 8), lambda i: (i, 0, 0, 0)),
        ],
        out_shape=[
            jax.ShapeDtypeStruct((b, 1), jnp.float32),
            jax.ShapeDtypeStruct((nb, 8, c), jnp.float32),
            jax.ShapeDtypeStruct((nb, c, 8, _BLK // 8), jnp.int32),
        ],
    )(task_embeddings, W1, b1.reshape(1, hh), W2, b2.reshape(1, 1),
      contributor_labels, ones8, onesc)

    agree = jnp.sum(agg[:, 0, :], axis=0)
    rel = agree * jnp.float32(1.0 / b)

    relt = jnp.broadcast_to(rel[:, None, None], (c, 8, _BLK // 8))
    t3 = t_out.reshape(nb, 8, _BLK // 8)
    relrow = jnp.broadcast_to(rel[None, :], (8, c))

    cur3, rel_b = pl.pallas_call(
        _stage2,
        grid=(nb,),
        in_specs=[
            pl.BlockSpec((1, c, 8, _BLK // 8), lambda i: (i, 0, 0, 0)),
            pl.BlockSpec((c, 8, _BLK // 8), lambda i: (0, 0, 0)),
            pl.BlockSpec((1, 8, _BLK // 8), lambda i: (i, 0, 0)),
            pl.BlockSpec((8, c), lambda i: (0, 0)),
        ],
        out_specs=[
            pl.BlockSpec((1, 8, _BLK // 8), lambda i: (i, 0, 0)),
            pl.BlockSpec((_BLK, c), lambda i: (i, 0)),
        ],
        out_shape=[
            jax.ShapeDtypeStruct((nb, 8, _BLK // 8), jnp.int32),
            jax.ShapeDtypeStruct((b, c), jnp.float32),
        ],
    )(labtr, relt, t3, relrow)

    return cur3.reshape(b), rel_b


# speedup vs baseline: 1.1390x; 1.1390x over previous
import jax
import jax.numpy as jnp
from jax.experimental import pallas as pl
from jax.experimental.pallas import tpu as pltpu

_BLK = 1024


def _fused(te_ref, w1_ref, b1_ref, w2_ref, b2_ref, lab_ref, ones8_ref,
           onesc_ref, ones128_ref, cur_ref, relb_ref, t_scr, agg_scr):
    i = pl.program_id(0)
    nb = pl.num_programs(0) // 2

    @pl.when(i < nb)
    def _phase_a():
        h = jnp.dot(te_ref[...], w1_ref[...], preferred_element_type=jnp.float32)
        h = jax.nn.silu(h + b1_ref[...])
        u = jnp.dot(h, w2_ref[...], preferred_element_type=jnp.float32)
        tval = 1.0 - jax.nn.sigmoid(u + b2_ref[...])
        t_scr[i] = tval.reshape(8, _BLK // 8)

        lab = lab_ref[...]
        c = lab.shape[1]
        labf1 = (lab == 1).astype(jnp.float32)
        count1 = jnp.dot(labf1, onesc_ref[...], preferred_element_type=jnp.float32)
        init = (count1 > (c - count1)).astype(jnp.int32)
        eqf = (lab == init).astype(jnp.float32)
        part = jnp.dot(ones8_ref[...], eqf, preferred_element_type=jnp.float32)

        @pl.when(i == 0)
        def _():
            agg_scr[...] = jnp.zeros_like(agg_scr)

        agg_scr[...] += part

    @pl.when(i >= nb)
    def _phase_b():
        lab = lab_ref[...]
        c = lab.shape[1]
        agree_row = agg_scr[0:1, :]
        relrow = agree_row * jnp.float32(1.0 / (nb * _BLK))
        rel_col = jnp.transpose(relrow, (1, 0))
        relmat = jnp.dot(rel_col, ones128_ref[...],
                         preferred_element_type=jnp.float32)

        t = t_scr[i - nb]
        lab3 = lab.reshape(8, _BLK // 8, c)
        labt = jnp.transpose(lab3, (2, 0, 1))
        nblk = 4
        blk = c // nblk

        s0 = None
        s1 = None
        for j in range(nblk):
            a1 = None
            a0 = None
            for k in range(blk):
                col = j * blk + k
                w = jnp.broadcast_to(relmat[col:col + 1, :], (8, _BLK // 8)) * t
                term1 = jnp.where(labt[col] == 1, w, 0.0)
                term0 = w - term1
                a1 = term1 if a1 is None else a1 + term1
                a0 = term0 if a0 is None else a0 + term0
            s1 = a1 if s1 is None else s1 + a1
            s0 = a0 if s0 is None else s0 + a0
        cur_ref[0] = (s1 > s0).astype(jnp.int32)

        relb_ref[...] = jnp.broadcast_to(relrow, relb_ref.shape)


def kernel(task_embeddings, contributor_ids, contributor_labels, W1, b1, W2, b2):
    del contributor_ids
    b, hidden = task_embeddings.shape
    c = contributor_labels.shape[1]
    hh = W1.shape[1]
    nb = b // _BLK

    ones8 = jnp.ones((8, _BLK), jnp.float32)
    onesc = jnp.ones((c, 1), jnp.float32)
    ones128 = jnp.ones((1, _BLK // 8), jnp.float32)

    cur3, rel_b = pl.pallas_call(
        _fused,
        grid=(2 * nb,),
        in_specs=[
            pl.BlockSpec((_BLK, hidden), lambda i, _n=nb: (jnp.minimum(i, _n - 1), 0)),
            pl.BlockSpec((hidden, hh), lambda i: (0, 0)),
            pl.BlockSpec((1, hh), lambda i: (0, 0)),
            pl.BlockSpec((hh, 1), lambda i: (0, 0)),
            pl.BlockSpec((1, 1), lambda i: (0, 0)),
            pl.BlockSpec((_BLK, c), lambda i, _n=nb: (jnp.where(i < _n, i, i - _n), 0)),
            pl.BlockSpec((8, _BLK), lambda i: (0, 0)),
            pl.BlockSpec((c, 1), lambda i: (0, 0)),
            pl.BlockSpec((1, _BLK // 8), lambda i: (0, 0)),
        ],
        out_specs=[
            pl.BlockSpec((1, 8, _BLK // 8),
                         lambda i, _n=nb: (jnp.where(i < _n, 0, i - _n), 0, 0)),
            pl.BlockSpec((_BLK, c), lambda i, _n=nb: (jnp.where(i < _n, 0, i - _n), 0)),
        ],
        out_shape=[
            jax.ShapeDtypeStruct((nb, 8, _BLK // 8), jnp.int32),
            jax.ShapeDtypeStruct((b, c), jnp.float32),
        ],
        scratch_shapes=[
            pltpu.VMEM((nb, 8, _BLK // 8), jnp.float32),
            pltpu.VMEM((8, c), jnp.float32),
        ],
    )(task_embeddings, W1, b1.reshape(1, hh), W2, b2.reshape(1, 1),
      contributor_labels, ones8, onesc, ones128)

    return cur3.reshape(b), rel_b


# speedup vs baseline: 1.1411x; 1.0018x over previous
import jax
import jax.numpy as jnp
from jax.experimental import pallas as pl
from jax.experimental.pallas import tpu as pltpu

_BLK = 1024


def _fused(te_ref, w1_ref, b1_ref, w2_ref, b2_ref, lab_ref, ones8_ref,
           onesc_ref, ones128_ref, cur_ref, relb_ref, t_scr, agg_scr):
    i = pl.program_id(0)
    nb = pl.num_programs(0) // 2

    @pl.when(i < nb)
    def _phase_a():
        h = jnp.dot(te_ref[...], w1_ref[...], preferred_element_type=jnp.float32)
        h = jax.nn.silu(h + b1_ref[...])
        u = jnp.dot(h, w2_ref[...], preferred_element_type=jnp.float32)
        tval = 1.0 - jax.nn.sigmoid(u + b2_ref[...])
        t_scr[i] = tval.reshape(8, _BLK // 8)

        lab = lab_ref[...]
        c = lab.shape[1]
        labf1 = (lab == 1).astype(jnp.float32)
        count1 = jnp.dot(labf1, onesc_ref[...], preferred_element_type=jnp.float32)
        init = (count1 > (c - count1)).astype(jnp.int32)
        eqf = (lab == init).astype(jnp.float32)
        part = jnp.dot(ones8_ref[...], eqf, preferred_element_type=jnp.float32)

        @pl.when(i == 0)
        def _():
            agg_scr[...] = jnp.zeros_like(agg_scr)

        agg_scr[...] += part

    @pl.when(i >= nb)
    def _phase_b():
        lab = lab_ref[...]
        c = lab.shape[1]
        agree_row = agg_scr[0:1, :]
        relrow = agree_row * jnp.float32(1.0 / (nb * _BLK))
        rel_col = jnp.transpose(relrow, (1, 0))
        relmat = jnp.broadcast_to(rel_col, (c, _BLK // 8))

        t = t_scr[i - nb]
        lab3 = lab.reshape(8, _BLK // 8, c)
        labt = jnp.transpose(lab3, (2, 0, 1))
        nblk = 4
        blk = c // nblk

        s0 = None
        s1 = None
        for j in range(nblk):
            a1 = None
            a0 = None
            for k in range(blk):
                col = j * blk + k
                w = jnp.broadcast_to(relmat[col:col + 1, :], (8, _BLK // 8)) * t
                term1 = jnp.where(labt[col] == 1, w, 0.0)
                term0 = w - term1
                a1 = term1 if a1 is None else a1 + term1
                a0 = term0 if a0 is None else a0 + term0
            s1 = a1 if s1 is None else s1 + a1
            s0 = a0 if s0 is None else s0 + a0
        cur_ref[0] = (s1 > s0).astype(jnp.int32)

        relb_ref[...] = jnp.broadcast_to(relrow, relb_ref.shape)


def kernel(task_embeddings, contributor_ids, contributor_labels, W1, b1, W2, b2):
    del contributor_ids
    b, hidden = task_embeddings.shape
    c = contributor_labels.shape[1]
    hh = W1.shape[1]
    nb = b // _BLK

    ones8 = jnp.ones((8, _BLK), jnp.float32)
    onesc = jnp.ones((c, 1), jnp.float32)
    ones128 = jnp.ones((1, _BLK // 8), jnp.float32)

    cur3, rel_b = pl.pallas_call(
        _fused,
        grid=(2 * nb,),
        in_specs=[
            pl.BlockSpec((_BLK, hidden), lambda i, _n=nb: (jnp.minimum(i, _n - 1), 0)),
            pl.BlockSpec((hidden, hh), lambda i: (0, 0)),
            pl.BlockSpec((1, hh), lambda i: (0, 0)),
            pl.BlockSpec((hh, 1), lambda i: (0, 0)),
            pl.BlockSpec((1, 1), lambda i: (0, 0)),
            pl.BlockSpec((_BLK, c), lambda i, _n=nb: (jnp.where(i < _n, i, i - _n), 0)),
            pl.BlockSpec((8, _BLK), lambda i: (0, 0)),
            pl.BlockSpec((c, 1), lambda i: (0, 0)),
            pl.BlockSpec((1, _BLK // 8), lambda i: (0, 0)),
        ],
        out_specs=[
            pl.BlockSpec((1, 8, _BLK // 8),
                         lambda i, _n=nb: (jnp.where(i < _n, 0, i - _n), 0, 0)),
            pl.BlockSpec((_BLK, c), lambda i, _n=nb: (jnp.where(i < _n, 0, i - _n), 0)),
        ],
        out_shape=[
            jax.ShapeDtypeStruct((nb, 8, _BLK // 8), jnp.int32),
            jax.ShapeDtypeStruct((b, c), jnp.float32),
        ],
        scratch_shapes=[
            pltpu.VMEM((nb, 8, _BLK // 8), jnp.float32),
            pltpu.VMEM((8, c), jnp.float32),
        ],
    )(task_embeddings, W1, b1.reshape(1, hh), W2, b2.reshape(1, 1),
      contributor_labels, ones8, onesc, ones128)

    return cur3.reshape(b), rel_b


# speedup vs baseline: 1.1479x; 1.0059x over previous
import jax
import jax.numpy as jnp
from jax.experimental import pallas as pl
from jax.experimental.pallas import tpu as pltpu

_BLK = 1024


def _fused(te_ref, w1_ref, b1_ref, w2_ref, b2_ref, lab_ref, ones8_ref,
           onesc_ref, cur_ref, relb_ref, t_scr, agg_scr):
    i = pl.program_id(0)
    nb = pl.num_programs(0) // 2

    @pl.when(i < nb)
    def _phase_a():
        h = jnp.dot(te_ref[...], w1_ref[...], preferred_element_type=jnp.float32)
        h = jax.nn.silu(h + b1_ref[...])
        u = jnp.dot(h, w2_ref[...], preferred_element_type=jnp.float32)
        tval = 1.0 - jax.nn.sigmoid(u + b2_ref[...])
        t_scr[i] = tval.reshape(8, _BLK // 8)

        lab = lab_ref[...]
        c = lab.shape[1]
        labf1 = (lab == 1).astype(jnp.float32)
        count1 = jnp.dot(labf1, onesc_ref[...], preferred_element_type=jnp.float32)
        init = (count1 > (c - count1)).astype(jnp.int32)
        eqf = (lab == init).astype(jnp.float32)
        part = jnp.dot(ones8_ref[...], eqf, preferred_element_type=jnp.float32)

        @pl.when(i == 0)
        def _():
            agg_scr[...] = jnp.zeros_like(agg_scr)

        agg_scr[...] += part

    @pl.when(i >= nb)
    def _phase_b():
        lab = lab_ref[...]
        c = lab.shape[1]
        agree_row = agg_scr[0:1, :]
        relrow = agree_row * jnp.float32(1.0 / (nb * _BLK))
        rel_col = jnp.transpose(relrow, (1, 0))
        relmat = jnp.broadcast_to(rel_col, (c, _BLK // 8))

        t = t_scr[i - nb]
        lab3 = lab.reshape(8, _BLK // 8, c)
        labt = jnp.transpose(lab3, (2, 0, 1))
        nblk = 4
        blk = c // nblk

        s0 = None
        s1 = None
        for j in range(nblk):
            a1 = None
            a0 = None
            for k in range(blk):
                col = j * blk + k
                w = jnp.broadcast_to(relmat[col:col + 1, :], (8, _BLK // 8)) * t
                term1 = jnp.where(labt[col] == 1, w, 0.0)
                term0 = w - term1
                a1 = term1 if a1 is None else a1 + term1
                a0 = term0 if a0 is None else a0 + term0
            s1 = a1 if s1 is None else s1 + a1
            s0 = a0 if s0 is None else s0 + a0
        cur_ref[0] = (s1 > s0).astype(jnp.int32)

        relb_ref[...] = jnp.broadcast_to(relrow, relb_ref.shape)


def kernel(task_embeddings, contributor_ids, contributor_labels, W1, b1, W2, b2):
    del contributor_ids
    b, hidden = task_embeddings.shape
    c = contributor_labels.shape[1]
    hh = W1.shape[1]
    nb = b // _BLK

    ones8 = jnp.ones((8, _BLK), jnp.float32)
    onesc = jnp.ones((c, 1), jnp.float32)

    cur3, rel_b = pl.pallas_call(
        _fused,
        grid=(2 * nb,),
        in_specs=[
            pl.BlockSpec((_BLK, hidden), lambda i, _n=nb: (jnp.minimum(i, _n - 1), 0)),
            pl.BlockSpec((hidden, hh), lambda i: (0, 0)),
            pl.BlockSpec((1, hh), lambda i: (0, 0)),
            pl.BlockSpec((hh, 1), lambda i: (0, 0)),
            pl.BlockSpec((1, 1), lambda i: (0, 0)),
            pl.BlockSpec((_BLK, c), lambda i, _n=nb: (jnp.where(i < _n, i, i - _n), 0)),
            pl.BlockSpec((8, _BLK), lambda i: (0, 0)),
            pl.BlockSpec((c, 1), lambda i: (0, 0)),
        ],
        out_specs=[
            pl.BlockSpec((1, 8, _BLK // 8),
                         lambda i, _n=nb: (jnp.where(i < _n, 0, i - _n), 0, 0)),
            pl.BlockSpec((_BLK, c), lambda i, _n=nb: (jnp.where(i < _n, 0, i - _n), 0)),
        ],
        out_shape=[
            jax.ShapeDtypeStruct((nb, 8, _BLK // 8), jnp.int32),
            jax.ShapeDtypeStruct((b, c), jnp.float32),
        ],
        scratch_shapes=[
            pltpu.VMEM((nb, 8, _BLK // 8), jnp.float32),
            pltpu.VMEM((8, c), jnp.float32),
        ],
    )(task_embeddings, W1, b1.reshape(1, hh), W2, b2.reshape(1, 1),
      contributor_labels, ones8, onesc)

    return cur3.reshape(b), rel_b
